# baseline (speedup 1.0000x reference)
import functools

import jax
import jax.numpy as jnp
from jax import lax
from jax.experimental import pallas as pl
from jax.experimental.pallas import tpu as pltpu
from jax.experimental.pallas import tpu_sc as plsc

_N = 10000
_E = 320000
_D = 128
_NC = 2
_NS = 16
_CHUNK = 128
_CPT = 80
_EPT = _CPT * _CHUNK
_REAL_PT = _E // (_NC * _NS)
_PAD_PT = _EPT - _REAL_PT
_NPAD = 10112
_DPAD = 16384
_RPT = _NPAD // _NS
_PHC = 40


def _sc_agg_body(x_hbm, src_hbm, dst_hbm, z2_hbm, z1_hbm, ones_hbm,
                 agg_out, dega_out, degb_out,
                 src_t, dst_t, rows0, rows1, ones_t, dbuf, acc, deg,
                 sem_g0, sem_g1):
    c = lax.axis_index("c")
    s = lax.axis_index("s")
    w = c * _NS + s

    pltpu.sync_copy(ones_hbm, ones_t)

    base = s * _RPT
    pltpu.sync_copy(z2_hbm, rows0)
    for kk in range(4):
        pltpu.sync_copy(rows0, acc.at[pl.ds(base + kk * 128, 128)])
    pltpu.sync_copy(rows0.at[pl.ds(0, _RPT - 512)],
                    acc.at[pl.ds(base + 512, _RPT - 512)])
    pltpu.sync_copy(z1_hbm, dbuf)
    pltpu.sync_copy(dbuf, deg.at[pl.ds(s * 1024, 1024)])
    plsc.subcore_barrier()

    for p in range(_CPT // _PHC):
        pltpu.sync_copy(src_hbm.at[w, pl.ds(p * _PHC, _PHC)], src_t)
        pltpu.sync_copy(dst_hbm.at[w, pl.ds(p * _PHC, _PHC)], dst_t)
        pltpu.async_copy(x_hbm.at[src_t.at[0]], rows0, sem_g0)
        pltpu.async_copy(x_hbm.at[src_t.at[1]], rows1, sem_g1)

        @pl.loop(0, _PHC // 2)
        def _edge_chunk(i):
            j0 = 2 * i
            for rows, sem_g, off in ((rows0, sem_g0, 0), (rows1, sem_g1, 1)):
                j = j0 + off
                pltpu.make_async_copy(x_hbm.at[src_t.at[j]], rows,
                                      sem_g).wait()
                pltpu.sync_copy(rows, acc.at[dst_t.at[j]], add=True)
                pltpu.sync_copy(ones_t, deg.at[dst_t.at[j]], add=True)

                @pl.when(j + 2 < _PHC)
                def _():
                    pltpu.async_copy(x_hbm.at[src_t.at[j + 2]], rows, sem_g)

    plsc.subcore_barrier()

    for kk in range(4):
        pltpu.sync_copy(acc.at[pl.ds(base + kk * 128, 128)], rows0)
        pltpu.sync_copy(rows0, agg_out.at[c, pl.ds(base + kk * 128, 128)])
    pltpu.sync_copy(acc.at[pl.ds(base + 512, _RPT - 512)],
                    rows0.at[pl.ds(0, _RPT - 512)])
    pltpu.sync_copy(rows0.at[pl.ds(0, _RPT - 512)],
                    agg_out.at[c, pl.ds(base + 512, _RPT - 512)])
    pltpu.sync_copy(deg.at[pl.ds(s * 1024, 1024)], dbuf)

    @pl.when(c == 0)
    def _():
        pltpu.sync_copy(dbuf, dega_out.at[pl.ds(s * 1024, 1024)])

    @pl.when(c == 1)
    def _():
        pltpu.sync_copy(dbuf, degb_out.at[pl.ds(s * 1024, 1024)])


_sc_agg = functools.partial(
    pl.kernel,
    out_type=(jax.ShapeDtypeStruct((_NC, _NPAD, _D), jnp.float32),
              jax.ShapeDtypeStruct((_DPAD,), jnp.float32),
              jax.ShapeDtypeStruct((_DPAD,), jnp.float32)),
    mesh=plsc.VectorSubcoreMesh(core_axis_name="c", subcore_axis_name="s"),
    scratch_types=[
        pltpu.VMEM((_PHC, _CHUNK), jnp.int32),
        pltpu.VMEM((_PHC, _CHUNK), jnp.int32),
        pltpu.VMEM((_CHUNK, _D), jnp.float32),
        pltpu.VMEM((_CHUNK, _D), jnp.float32),
        pltpu.VMEM((_CHUNK,), jnp.float32),
        pltpu.VMEM((1024,), jnp.float32),
        pltpu.VMEM_SHARED((_NPAD, _D), jnp.float32),
        pltpu.VMEM_SHARED((_DPAD,), jnp.float32),
        pltpu.SemaphoreType.DMA,
        pltpu.SemaphoreType.DMA,
    ],
)(_sc_agg_body)


def _tc_mlp_body(agg_ref, dega_ref, degb_ref, w1, b1, wh, bh, wo, bo, out_ref):
    a = agg_ref[0] + agg_ref[1]
    d = dega_ref[...] + degb_ref[...]
    h = a / jnp.maximum(d, 1.0)
    h = jnp.maximum(
        jnp.dot(h, w1[...], preferred_element_type=jnp.float32) + b1[...], 0.0)
    h = jnp.maximum(
        jnp.dot(h, wh[...], preferred_element_type=jnp.float32) + bh[...], 0.0)
    out_ref[...] = (
        jnp.dot(h, wo[...], preferred_element_type=jnp.float32) + bo[...])


_ROWS_BLK = 400
_tc_mlp = pl.pallas_call(
    _tc_mlp_body,
    grid=(_N // _ROWS_BLK,),
    in_specs=[
        pl.BlockSpec((_NC, _ROWS_BLK, _D), lambda i: (0, i, 0)),
        pl.BlockSpec((_ROWS_BLK, 1), lambda i: (i, 0)),
        pl.BlockSpec((_ROWS_BLK, 1), lambda i: (i, 0)),
        pl.BlockSpec((_D, _D), lambda i: (0, 0)),
        pl.BlockSpec((1, _D), lambda i: (0, 0)),
        pl.BlockSpec((_D, _D), lambda i: (0, 0)),
        pl.BlockSpec((1, _D), lambda i: (0, 0)),
        pl.BlockSpec((_D, _D), lambda i: (0, 0)),
        pl.BlockSpec((1, _D), lambda i: (0, 0)),
    ],
    out_specs=pl.BlockSpec((_ROWS_BLK, _D), lambda i: (i, 0)),
    out_shape=jax.ShapeDtypeStruct((_N, _D), jnp.float32),
)


def kernel(x, edge_index, W1, b1, Wh, bh, Wo, bo):
    src = edge_index[0].astype(jnp.int32)
    dst = edge_index[1].astype(jnp.int32)
    nw = _NC * _NS
    npad = nw * _EPT - _E
    pad_src = jnp.zeros((npad,), jnp.int32)
    pad_dst = _N + (jnp.arange(npad, dtype=jnp.int32) % (_NPAD - _N))
    src_p = jnp.concatenate([src, pad_src]).reshape(nw, _CPT, _CHUNK)
    dst_p = jnp.concatenate([dst, pad_dst]).reshape(nw, _CPT, _CHUNK)
    zeros2 = jnp.zeros((_CHUNK, _D), jnp.float32)
    zeros1 = jnp.zeros((1024,), jnp.float32)
    ones = jnp.ones((_CHUNK,), jnp.float32)

    agg2, dega, degb = _sc_agg(x, src_p, dst_p, zeros2, zeros1, ones)
    y = _tc_mlp(agg2, dega.reshape(_DPAD, 1), degb.reshape(_DPAD, 1),
                W1, b1.reshape(1, _D), Wh, bh.reshape(1, _D),
                Wo, bo.reshape(1, _D))
    return y


# speedup vs baseline: 3.0071x; 3.0071x over previous
import functools

import jax
import jax.numpy as jnp
from jax import lax
from jax.experimental import pallas as pl
from jax.experimental.pallas import tpu as pltpu
from jax.experimental.pallas import tpu_sc as plsc

_N = 10000
_E = 320000
_D = 128
_NC = 2
_NS = 16
_CHUNK = 128
_CPT = 80
_EPT = _CPT * _CHUNK
_REAL_PT = _E // (_NC * _NS)
_PAD_PT = _EPT - _REAL_PT
_NPAD = 10112
_DPAD = 16384
_RPT = _NPAD // _NS
_PHC = 40


def _sc_agg_body(x_hbm, src_hbm, dst_hbm, z2_hbm, z1_hbm, ones_hbm,
                 agg_out, dega_out, degb_out,
                 src_t, dst_t, rows0, rows1, ones_t, dbuf, acc, deg,
                 sem_g0, sem_g1):
    c = lax.axis_index("c")
    s = lax.axis_index("s")
    w = c * _NS + s

    pltpu.sync_copy(ones_hbm, ones_t)

    base = s * _RPT
    pltpu.sync_copy(z2_hbm, rows0)
    for kk in range(4):
        pltpu.sync_copy(rows0, acc.at[pl.ds(base + kk * 128, 128)])
    pltpu.sync_copy(rows0.at[pl.ds(0, _RPT - 512)],
                    acc.at[pl.ds(base + 512, _RPT - 512)])
    pltpu.sync_copy(z1_hbm, dbuf)
    pltpu.sync_copy(dbuf, deg.at[pl.ds(s * 1024, 1024)])
    plsc.subcore_barrier()

    for p in range(_CPT // _PHC):
        pltpu.sync_copy(src_hbm.at[w, pl.ds(p * _PHC, _PHC)], src_t)
        pltpu.sync_copy(dst_hbm.at[w, pl.ds(p * _PHC, _PHC)], dst_t)
        pltpu.async_copy(x_hbm.at[src_t.at[0]], rows0, sem_g0)
        pltpu.async_copy(x_hbm.at[src_t.at[1]], rows1, sem_g1)

        @pl.loop(0, _PHC // 2)
        def _edge_chunk(i):
            j0 = 2 * i
            for rows, sem_g, off in ((rows0, sem_g0, 0), (rows1, sem_g1, 1)):
                j = j0 + off
                pltpu.make_async_copy(x_hbm.at[src_t.at[j]], rows,
                                      sem_g).wait()
                pltpu.sync_copy(rows, acc.at[dst_t.at[j]], add=True)
                pltpu.sync_copy(ones_t, deg.at[dst_t.at[j]], add=True)

                @pl.when(j + 2 < _PHC)
                def _():
                    pltpu.async_copy(x_hbm.at[src_t.at[j + 2]], rows, sem_g)

    plsc.subcore_barrier()

    for kk in range(4):
        pltpu.sync_copy(acc.at[pl.ds(base + kk * 128, 128)], rows0)
        pltpu.sync_copy(rows0, agg_out.at[c, pl.ds(base + kk * 128, 128)])
    pltpu.sync_copy(acc.at[pl.ds(base + 512, _RPT - 512)],
                    rows0.at[pl.ds(0, _RPT - 512)])
    pltpu.sync_copy(rows0.at[pl.ds(0, _RPT - 512)],
                    agg_out.at[c, pl.ds(base + 512, _RPT - 512)])
    pltpu.sync_copy(deg.at[pl.ds(s * 1024, 1024)], dbuf)

    @pl.when(c == 0)
    def _():
        pltpu.sync_copy(dbuf, dega_out.at[pl.ds(s * 1024, 1024)])

    @pl.when(c == 1)
    def _():
        pltpu.sync_copy(dbuf, degb_out.at[pl.ds(s * 1024, 1024)])


_sc_agg = functools.partial(
    pl.kernel,
    out_type=(jax.ShapeDtypeStruct((_NC, _NPAD, _D), jnp.float32),
              jax.ShapeDtypeStruct((_DPAD,), jnp.float32),
              jax.ShapeDtypeStruct((_DPAD,), jnp.float32)),
    mesh=plsc.VectorSubcoreMesh(core_axis_name="c", subcore_axis_name="s"),
    scratch_types=[
        pltpu.VMEM((_PHC, _CHUNK), jnp.int32),
        pltpu.VMEM((_PHC, _CHUNK), jnp.int32),
        pltpu.VMEM((_CHUNK, _D), jnp.float32),
        pltpu.VMEM((_CHUNK, _D), jnp.float32),
        pltpu.VMEM((_CHUNK,), jnp.float32),
        pltpu.VMEM((1024,), jnp.float32),
        pltpu.VMEM_SHARED((_NPAD, _D), jnp.float32),
        pltpu.VMEM_SHARED((_DPAD,), jnp.float32),
        pltpu.SemaphoreType.DMA,
        pltpu.SemaphoreType.DMA,
    ],
)(_sc_agg_body)


def _tc_mlp_body(agg_ref, dega_ref, degb_ref, w1, b1, wh, bh, wo, bo, out_ref):
    a = agg_ref[0] + agg_ref[1]
    d = dega_ref[...] + degb_ref[...]
    h = a / jnp.maximum(d, 1.0)
    h = jnp.maximum(
        jnp.dot(h, w1[...], preferred_element_type=jnp.float32) + b1[...], 0.0)
    h = jnp.maximum(
        jnp.dot(h, wh[...], preferred_element_type=jnp.float32) + bh[...], 0.0)
    out_ref[...] = (
        jnp.dot(h, wo[...], preferred_element_type=jnp.float32) + bo[...])


_ROWS_BLK = 400
_tc_mlp = pl.pallas_call(
    _tc_mlp_body,
    grid=(_N // _ROWS_BLK,),
    in_specs=[
        pl.BlockSpec((_NC, _ROWS_BLK, _D), lambda i: (0, i, 0)),
        pl.BlockSpec((_ROWS_BLK, 1), lambda i: (i, 0)),
        pl.BlockSpec((_ROWS_BLK, 1), lambda i: (i, 0)),
        pl.BlockSpec((_D, _D), lambda i: (0, 0)),
        pl.BlockSpec((1, _D), lambda i: (0, 0)),
        pl.BlockSpec((_D, _D), lambda i: (0, 0)),
        pl.BlockSpec((1, _D), lambda i: (0, 0)),
        pl.BlockSpec((_D, _D), lambda i: (0, 0)),
        pl.BlockSpec((1, _D), lambda i: (0, 0)),
    ],
    out_specs=pl.BlockSpec((_ROWS_BLK, _D), lambda i: (i, 0)),
    out_shape=jax.ShapeDtypeStruct((_N, _D), jnp.float32),
)


def kernel(x, edge_index, W1, b1, Wh, bh, Wo, bo):
    src = edge_index[0].astype(jnp.int32)
    dst = edge_index[1].astype(jnp.int32)
    nw = _NC * _NS
    npad = nw * _EPT - _E
    pad_src = jnp.arange(npad, dtype=jnp.int32) % _N
    pad_dst = _N + (jnp.arange(npad, dtype=jnp.int32) % (_NPAD - _N))
    src_p = jnp.concatenate([src, pad_src]).reshape(nw, _CPT, _CHUNK)
    dst_p = jnp.concatenate([dst, pad_dst]).reshape(nw, _CPT, _CHUNK)
    zeros2 = jnp.zeros((_CHUNK, _D), jnp.float32)
    zeros1 = jnp.zeros((1024,), jnp.float32)
    ones = jnp.ones((_CHUNK,), jnp.float32)

    agg2, dega, degb = _sc_agg(x, src_p, dst_p, zeros2, zeros1, ones)
    y = _tc_mlp(agg2, dega.reshape(_DPAD, 1), degb.reshape(_DPAD, 1),
                W1, b1.reshape(1, _D), Wh, bh.reshape(1, _D),
                Wo, bo.reshape(1, _D))
    return y


# speedup vs baseline: 3.0463x; 1.0130x over previous
import functools

import jax
import jax.numpy as jnp
from jax import lax
from jax.experimental import pallas as pl
from jax.experimental.pallas import tpu as pltpu
from jax.experimental.pallas import tpu_sc as plsc

_N = 10000
_E = 320000
_D = 128
_NC = 2
_NS = 16
_CHUNK = 128
_CPT = 80
_EPT = _CPT * _CHUNK
_REAL_PT = _E // (_NC * _NS)
_PAD_PT = _EPT - _REAL_PT
_NPAD = 10112
_DPAD = 16384
_RPT = _NPAD // _NS
_PHC = 40


def _sc_agg_body(x_hbm, e3_hbm, tsrc_hbm, tdst_hbm, z2_hbm, z1_hbm, ones_hbm,
                 agg_out, dega_out, degb_out,
                 src_t, dst_t, rows0, rows1, ones_t, dbuf, acc, deg,
                 sem_g0, sem_g1):
    c = lax.axis_index("c")
    s = lax.axis_index("s")
    w = c * _NS + s

    pltpu.sync_copy(ones_hbm, ones_t)

    base = s * _RPT
    pltpu.sync_copy(z2_hbm, rows0)
    for kk in range(4):
        pltpu.sync_copy(rows0, acc.at[pl.ds(base + kk * 128, 128)])
    pltpu.sync_copy(rows0.at[pl.ds(0, _RPT - 512)],
                    acc.at[pl.ds(base + 512, _RPT - 512)])
    pltpu.sync_copy(z1_hbm, dbuf)
    pltpu.sync_copy(dbuf, deg.at[pl.ds(s * 1024, 1024)])
    plsc.subcore_barrier()

    for p in range(_CPT // _PHC):
        @pl.when(w < _NC * _NS - 1)
        def _():
            pltpu.sync_copy(e3_hbm.at[0, pl.ds(w * _CPT + p * _PHC, _PHC)],
                            src_t)
            pltpu.sync_copy(e3_hbm.at[1, pl.ds(w * _CPT + p * _PHC, _PHC)],
                            dst_t)

        @pl.when(w == _NC * _NS - 1)
        def _():
            pltpu.sync_copy(tsrc_hbm.at[pl.ds(p * _PHC, _PHC)], src_t)
            pltpu.sync_copy(tdst_hbm.at[pl.ds(p * _PHC, _PHC)], dst_t)
        pltpu.async_copy(x_hbm.at[src_t.at[0]], rows0, sem_g0)
        pltpu.async_copy(x_hbm.at[src_t.at[1]], rows1, sem_g1)

        @pl.loop(0, _PHC // 2)
        def _edge_chunk(i):
            j0 = 2 * i
            for rows, sem_g, off in ((rows0, sem_g0, 0), (rows1, sem_g1, 1)):
                j = j0 + off
                pltpu.make_async_copy(x_hbm.at[src_t.at[j]], rows,
                                      sem_g).wait()
                pltpu.sync_copy(rows, acc.at[dst_t.at[j]], add=True)
                pltpu.sync_copy(ones_t, deg.at[dst_t.at[j]], add=True)

                @pl.when(j + 2 < _PHC)
                def _():
                    pltpu.async_copy(x_hbm.at[src_t.at[j + 2]], rows, sem_g)

    plsc.subcore_barrier()

    for kk in range(4):
        pltpu.sync_copy(acc.at[pl.ds(base + kk * 128, 128)], rows0)
        pltpu.sync_copy(rows0, agg_out.at[c, pl.ds(base + kk * 128, 128)])
    pltpu.sync_copy(acc.at[pl.ds(base + 512, _RPT - 512)],
                    rows0.at[pl.ds(0, _RPT - 512)])
    pltpu.sync_copy(rows0.at[pl.ds(0, _RPT - 512)],
                    agg_out.at[c, pl.ds(base + 512, _RPT - 512)])
    pltpu.sync_copy(deg.at[pl.ds(s * 1024, 1024)], dbuf)

    @pl.when(c == 0)
    def _():
        pltpu.sync_copy(dbuf, dega_out.at[pl.ds(s * 1024, 1024)])

    @pl.when(c == 1)
    def _():
        pltpu.sync_copy(dbuf, degb_out.at[pl.ds(s * 1024, 1024)])


_sc_agg = functools.partial(
    pl.kernel,
    out_type=(jax.ShapeDtypeStruct((_NC, _NPAD, _D), jnp.float32),
              jax.ShapeDtypeStruct((_DPAD,), jnp.float32),
              jax.ShapeDtypeStruct((_DPAD,), jnp.float32)),
    mesh=plsc.VectorSubcoreMesh(core_axis_name="c", subcore_axis_name="s"),
    scratch_types=[
        pltpu.VMEM((_PHC, _CHUNK), jnp.int32),
        pltpu.VMEM((_PHC, _CHUNK), jnp.int32),
        pltpu.VMEM((_CHUNK, _D), jnp.float32),
        pltpu.VMEM((_CHUNK, _D), jnp.float32),
        pltpu.VMEM((_CHUNK,), jnp.float32),
        pltpu.VMEM((1024,), jnp.float32),
        pltpu.VMEM_SHARED((_NPAD, _D), jnp.float32),
        pltpu.VMEM_SHARED((_DPAD,), jnp.float32),
        pltpu.SemaphoreType.DMA,
        pltpu.SemaphoreType.DMA,
    ],
)(_sc_agg_body)


def _tc_mlp_body(agg_ref, dd_ref, w1, b1, wh, bh, wo, bo, out_ref):
    a = agg_ref[0] + agg_ref[1]
    h = a / jnp.maximum(dd_ref[...], 1.0)
    h = jnp.maximum(
        jnp.dot(h, w1[...], preferred_element_type=jnp.float32) + b1[...], 0.0)
    h = jnp.maximum(
        jnp.dot(h, wh[...], preferred_element_type=jnp.float32) + bh[...], 0.0)
    out_ref[...] = (
        jnp.dot(h, wo[...], preferred_element_type=jnp.float32) + bo[...])


_ROWS_BLK = 400
_tc_mlp = pl.pallas_call(
    _tc_mlp_body,
    grid=(_N // _ROWS_BLK,),
    in_specs=[
        pl.BlockSpec((_NC, _ROWS_BLK, _D), lambda i: (0, i, 0)),
        pl.BlockSpec((_ROWS_BLK, _D), lambda i: (i, 0)),
        pl.BlockSpec((_D, _D), lambda i: (0, 0)),
        pl.BlockSpec((1, _D), lambda i: (0, 0)),
        pl.BlockSpec((_D, _D), lambda i: (0, 0)),
        pl.BlockSpec((1, _D), lambda i: (0, 0)),
        pl.BlockSpec((_D, _D), lambda i: (0, 0)),
        pl.BlockSpec((1, _D), lambda i: (0, 0)),
    ],
    out_specs=pl.BlockSpec((_ROWS_BLK, _D), lambda i: (i, 0)),
    out_shape=jax.ShapeDtypeStruct((_N, _D), jnp.float32),
)


def kernel(x, edge_index, W1, b1, Wh, bh, Wo, bo):
    src = edge_index[0].astype(jnp.int32)
    dst = edge_index[1].astype(jnp.int32)
    nw = _NC * _NS
    e3 = edge_index.astype(jnp.int32).reshape(2, _E // _CHUNK, _CHUNK)
    ntail = _E - (nw - 1) * _EPT
    npad = _EPT - ntail
    pad_src = jnp.arange(npad, dtype=jnp.int32) % _N
    pad_dst = _N + (jnp.arange(npad, dtype=jnp.int32) % (_NPAD - _N))
    tsrc = jnp.concatenate([src[-ntail:], pad_src]).reshape(_CPT, _CHUNK)
    tdst = jnp.concatenate([dst[-ntail:], pad_dst]).reshape(_CPT, _CHUNK)
    zeros2 = jnp.zeros((_CHUNK, _D), jnp.float32)
    zeros1 = jnp.zeros((1024,), jnp.float32)
    ones = jnp.ones((_CHUNK,), jnp.float32)

    agg2, dega, degb = _sc_agg(x, e3, tsrc, tdst, zeros2, zeros1, ones)
    dd = jnp.broadcast_to((dega + degb)[:_N, None], (_N, _D))
    y = _tc_mlp(agg2, dd,
                W1, b1.reshape(1, _D), Wh, bh.reshape(1, _D),
                Wo, bo.reshape(1, _D))
    return y


# speedup vs baseline: 3.5010x; 1.1493x over previous
import functools

import jax
import jax.numpy as jnp
from jax import lax
from jax.experimental import pallas as pl
from jax.experimental.pallas import tpu as pltpu
from jax.experimental.pallas import tpu_sc as plsc

_N = 10000
_E = 320000
_D = 128
_NC = 2
_NS = 16
_CHUNK = 128
_CPT = 80
_EPT = _CPT * _CHUNK
_REAL_PT = _E // (_NC * _NS)
_PAD_PT = _EPT - _REAL_PT
_NPAD = 10112
_DPAD = 16384
_RPT = _NPAD // _NS
_PHC = 40


def _sc_agg_body(x_hbm, e3_hbm, tsrc_hbm, tdst_hbm, z2_hbm, z1_hbm, ones_hbm,
                 agg_out, dega_out, degb_out,
                 src_t, dst_t, rows0, rows1, ones_t, dbuf, acc, deg,
                 sem_g0, sem_g1):
    c = lax.axis_index("c")
    s = lax.axis_index("s")
    w = c * _NS + s

    pltpu.sync_copy(ones_hbm, ones_t)

    base = s * _RPT
    pltpu.sync_copy(z2_hbm, rows0)
    for kk in range(4):
        pltpu.sync_copy(rows0, acc.at[pl.ds(base + kk * 128, 128)])
    pltpu.sync_copy(rows0.at[pl.ds(0, _RPT - 512)],
                    acc.at[pl.ds(base + 512, _RPT - 512)])
    pltpu.sync_copy(z1_hbm, dbuf)
    pltpu.sync_copy(dbuf, deg.at[pl.ds(s * 1024, 1024)])
    plsc.subcore_barrier()

    for p in range(_CPT // _PHC):
        @pl.when(w < _NC * _NS - 1)
        def _():
            pltpu.sync_copy(e3_hbm.at[0, pl.ds(w * _CPT + p * _PHC, _PHC)],
                            src_t)
            pltpu.sync_copy(e3_hbm.at[1, pl.ds(w * _CPT + p * _PHC, _PHC)],
                            dst_t)

        @pl.when(w == _NC * _NS - 1)
        def _():
            pltpu.sync_copy(tsrc_hbm.at[pl.ds(p * _PHC, _PHC)], src_t)
            pltpu.sync_copy(tdst_hbm.at[pl.ds(p * _PHC, _PHC)], dst_t)
        pltpu.async_copy(x_hbm.at[src_t.at[0]], rows0, sem_g0)
        pltpu.async_copy(x_hbm.at[src_t.at[1]], rows1, sem_g1)

        @pl.loop(0, _PHC // 2)
        def _edge_chunk(i):
            j0 = 2 * i
            for rows, sem_g, off in ((rows0, sem_g0, 0), (rows1, sem_g1, 1)):
                j = j0 + off
                pltpu.make_async_copy(x_hbm.at[src_t.at[j]], rows,
                                      sem_g).wait()
                pltpu.sync_copy(rows, acc.at[dst_t.at[j]], add=True)
                pltpu.sync_copy(ones_t, deg.at[dst_t.at[j]], add=True)

                @pl.when(j + 2 < _PHC)
                def _():
                    pltpu.async_copy(x_hbm.at[src_t.at[j + 2]], rows, sem_g)

    plsc.subcore_barrier()

    for kk in range(4):
        pltpu.sync_copy(acc.at[pl.ds(base + kk * 128, 128)], rows0)
        pltpu.sync_copy(rows0, agg_out.at[c, pl.ds(base + kk * 128, 128)])
    pltpu.sync_copy(acc.at[pl.ds(base + 512, _RPT - 512)],
                    rows0.at[pl.ds(0, _RPT - 512)])
    pltpu.sync_copy(rows0.at[pl.ds(0, _RPT - 512)],
                    agg_out.at[c, pl.ds(base + 512, _RPT - 512)])
    pltpu.sync_copy(deg.at[pl.ds(s * 1024, 1024)], dbuf)

    @pl.when(c == 0)
    def _():
        pltpu.sync_copy(dbuf, dega_out.at[pl.ds(s * 1024, 1024)])

    @pl.when(c == 1)
    def _():
        pltpu.sync_copy(dbuf, degb_out.at[pl.ds(s * 1024, 1024)])


_sc_agg = functools.partial(
    pl.kernel,
    out_type=(jax.ShapeDtypeStruct((_NC, _NPAD, _D), jnp.float32),
              jax.ShapeDtypeStruct((_DPAD,), jnp.float32),
              jax.ShapeDtypeStruct((_DPAD,), jnp.float32)),
    mesh=plsc.VectorSubcoreMesh(core_axis_name="c", subcore_axis_name="s"),
    scratch_types=[
        pltpu.VMEM((_PHC, _CHUNK), jnp.int32),
        pltpu.VMEM((_PHC, _CHUNK), jnp.int32),
        pltpu.VMEM((_CHUNK, _D), jnp.float32),
        pltpu.VMEM((_CHUNK, _D), jnp.float32),
        pltpu.VMEM((_CHUNK,), jnp.float32),
        pltpu.VMEM((1024,), jnp.float32),
        pltpu.VMEM_SHARED((_NPAD, _D), jnp.float32),
        pltpu.VMEM_SHARED((_DPAD,), jnp.float32),
        pltpu.SemaphoreType.DMA,
        pltpu.SemaphoreType.DMA,
    ],
)(_sc_agg_body)


def _tc_mlp_body(agg_ref, dd_ref, w1, b1, wh, bh, wo, bo, out_ref):
    a = agg_ref[0] + agg_ref[1]
    h = a / jnp.maximum(dd_ref[...], 1.0)
    h = jnp.maximum(
        jnp.dot(h, w1[...], preferred_element_type=jnp.float32) + b1[...], 0.0)
    h = jnp.maximum(
        jnp.dot(h, wh[...], preferred_element_type=jnp.float32) + bh[...], 0.0)
    out_ref[...] = (
        jnp.dot(h, wo[...], preferred_element_type=jnp.float32) + bo[...])


_ROWS_BLK = 2000
_tc_mlp = pl.pallas_call(
    _tc_mlp_body,
    grid=(_N // _ROWS_BLK,),
    in_specs=[
        pl.BlockSpec((_NC, _ROWS_BLK, _D), lambda i: (0, i, 0)),
        pl.BlockSpec((_ROWS_BLK, _D), lambda i: (i, 0)),
        pl.BlockSpec((_D, _D), lambda i: (0, 0)),
        pl.BlockSpec((1, _D), lambda i: (0, 0)),
        pl.BlockSpec((_D, _D), lambda i: (0, 0)),
        pl.BlockSpec((1, _D), lambda i: (0, 0)),
        pl.BlockSpec((_D, _D), lambda i: (0, 0)),
        pl.BlockSpec((1, _D), lambda i: (0, 0)),
    ],
    out_specs=pl.BlockSpec((_ROWS_BLK, _D), lambda i: (i, 0)),
    out_shape=jax.ShapeDtypeStruct((_N, _D), jnp.float32),
)


def kernel(x, edge_index, W1, b1, Wh, bh, Wo, bo):
    nw = _NC * _NS
    e3 = edge_index.astype(jnp.int32).reshape(2, _E // _CHUNK, _CHUNK)
    ntail = _E - (nw - 1) * _EPT
    npad = _EPT - ntail
    pad_src = jnp.arange(npad, dtype=jnp.int32) % _N
    pad_dst = _N + (jnp.arange(npad, dtype=jnp.int32) % (_NPAD - _N))
    tail = edge_index[:, _E - ntail:].astype(jnp.int32)
    tsrc = jnp.concatenate([tail[0], pad_src]).reshape(_CPT, _CHUNK)
    tdst = jnp.concatenate([tail[1], pad_dst]).reshape(_CPT, _CHUNK)
    zeros2 = jnp.zeros((_CHUNK, _D), jnp.float32)
    zeros1 = jnp.zeros((1024,), jnp.float32)
    ones = jnp.ones((_CHUNK,), jnp.float32)

    agg2, dega, degb = _sc_agg(x, e3, tsrc, tdst, zeros2, zeros1, ones)
    dd = jnp.broadcast_to((dega + degb)[:_N, None], (_N, _D))
    y = _tc_mlp(agg2, dd,
                W1, b1.reshape(1, _D), Wh, bh.reshape(1, _D),
                Wo, bo.reshape(1, _D))
    return y


# speedup vs baseline: 3.5208x; 1.0057x over previous
import functools

import jax
import jax.numpy as jnp
from jax import lax
from jax.experimental import pallas as pl
from jax.experimental.pallas import tpu as pltpu
from jax.experimental.pallas import tpu_sc as plsc

_N = 10000
_E = 320000
_D = 128
_NC = 2
_NS = 16
_CHUNK = 128
_CPT = 80
_EPT = _CPT * _CHUNK
_REAL_PT = _E // (_NC * _NS)
_PAD_PT = _EPT - _REAL_PT
_NPAD = 10112
_DPAD = 16384
_RPT = _NPAD // _NS
_PHC = 40


def _sc_agg_body(x_hbm, e3_hbm, tsrc_hbm, tdst_hbm, z2_hbm, z1_hbm, ones_hbm,
                 agg_out, dega_out, degb_out,
                 src_t, dst_t, rows0, rows1, ones_t, dbuf, acc, deg,
                 sem_g0, sem_g1, sem_d0, sem_d1):
    c = lax.axis_index("c")
    s = lax.axis_index("s")
    w = c * _NS + s

    pltpu.sync_copy(ones_hbm, ones_t)

    base = s * _RPT
    pltpu.sync_copy(z2_hbm, rows0)
    for kk in range(4):
        pltpu.sync_copy(rows0, acc.at[pl.ds(base + kk * 128, 128)])
    pltpu.sync_copy(rows0.at[pl.ds(0, _RPT - 512)],
                    acc.at[pl.ds(base + 512, _RPT - 512)])
    pltpu.sync_copy(z1_hbm, dbuf)
    pltpu.sync_copy(dbuf, deg.at[pl.ds(s * 1024, 1024)])
    plsc.subcore_barrier()

    for p in range(_CPT // _PHC):
        @pl.when(w < _NC * _NS - 1)
        def _():
            pltpu.sync_copy(e3_hbm.at[0, pl.ds(w * _CPT + p * _PHC, _PHC)],
                            src_t)
            pltpu.sync_copy(e3_hbm.at[1, pl.ds(w * _CPT + p * _PHC, _PHC)],
                            dst_t)

        @pl.when(w == _NC * _NS - 1)
        def _():
            pltpu.sync_copy(tsrc_hbm.at[pl.ds(p * _PHC, _PHC)], src_t)
            pltpu.sync_copy(tdst_hbm.at[pl.ds(p * _PHC, _PHC)], dst_t)
        pltpu.async_copy(x_hbm.at[src_t.at[0]], rows0, sem_g0)
        pltpu.async_copy(x_hbm.at[src_t.at[1]], rows1, sem_g1)

        @pl.loop(0, _PHC // 2)
        def _edge_chunk(i):
            j0 = 2 * i
            for rows, sem_g, sem_d, off in ((rows0, sem_g0, sem_d0, 0),
                                            (rows1, sem_g1, sem_d1, 1)):
                j = j0 + off
                pltpu.make_async_copy(x_hbm.at[src_t.at[j]], rows,
                                      sem_g).wait()

                @pl.when(j >= 2)
                def _():
                    pltpu.make_async_copy(ones_t, deg.at[dst_t.at[j - 2]],
                                          sem_d).wait()

                pltpu.async_copy(ones_t, deg.at[dst_t.at[j]], sem_d, add=True)
                pltpu.sync_copy(rows, acc.at[dst_t.at[j]], add=True)

                @pl.when(j + 2 < _PHC)
                def _():
                    pltpu.async_copy(x_hbm.at[src_t.at[j + 2]], rows, sem_g)

        pltpu.make_async_copy(ones_t, deg.at[dst_t.at[_PHC - 2]],
                              sem_d0).wait()
        pltpu.make_async_copy(ones_t, deg.at[dst_t.at[_PHC - 1]],
                              sem_d1).wait()

    plsc.subcore_barrier()

    for kk in range(4):
        pltpu.sync_copy(acc.at[pl.ds(base + kk * 128, 128)], rows0)
        pltpu.sync_copy(rows0, agg_out.at[c, pl.ds(base + kk * 128, 128)])
    pltpu.sync_copy(acc.at[pl.ds(base + 512, _RPT - 512)],
                    rows0.at[pl.ds(0, _RPT - 512)])
    pltpu.sync_copy(rows0.at[pl.ds(0, _RPT - 512)],
                    agg_out.at[c, pl.ds(base + 512, _RPT - 512)])
    pltpu.sync_copy(deg.at[pl.ds(s * 1024, 1024)], dbuf)

    @pl.when(c == 0)
    def _():
        pltpu.sync_copy(dbuf, dega_out.at[pl.ds(s * 1024, 1024)])

    @pl.when(c == 1)
    def _():
        pltpu.sync_copy(dbuf, degb_out.at[pl.ds(s * 1024, 1024)])


_sc_agg = functools.partial(
    pl.kernel,
    out_type=(jax.ShapeDtypeStruct((_NC, _NPAD, _D), jnp.float32),
              jax.ShapeDtypeStruct((_DPAD,), jnp.float32),
              jax.ShapeDtypeStruct((_DPAD,), jnp.float32)),
    mesh=plsc.VectorSubcoreMesh(core_axis_name="c", subcore_axis_name="s"),
    scratch_types=[
        pltpu.VMEM((_PHC, _CHUNK), jnp.int32),
        pltpu.VMEM((_PHC, _CHUNK), jnp.int32),
        pltpu.VMEM((_CHUNK, _D), jnp.float32),
        pltpu.VMEM((_CHUNK, _D), jnp.float32),
        pltpu.VMEM((_CHUNK,), jnp.float32),
        pltpu.VMEM((1024,), jnp.float32),
        pltpu.VMEM_SHARED((_NPAD, _D), jnp.float32),
        pltpu.VMEM_SHARED((_DPAD,), jnp.float32),
        pltpu.SemaphoreType.DMA,
        pltpu.SemaphoreType.DMA,
        pltpu.SemaphoreType.DMA,
        pltpu.SemaphoreType.DMA,
    ],
)(_sc_agg_body)


def _tc_mlp_body(agg_ref, dd_ref, w1, b1, wh, bh, wo, bo, out_ref):
    a = agg_ref[0] + agg_ref[1]
    h = a / jnp.maximum(dd_ref[...], 1.0)
    h = jnp.maximum(
        jnp.dot(h, w1[...], preferred_element_type=jnp.float32) + b1[...], 0.0)
    h = jnp.maximum(
        jnp.dot(h, wh[...], preferred_element_type=jnp.float32) + bh[...], 0.0)
    out_ref[...] = (
        jnp.dot(h, wo[...], preferred_element_type=jnp.float32) + bo[...])


_ROWS_BLK = 2000
_tc_mlp = pl.pallas_call(
    _tc_mlp_body,
    grid=(_N // _ROWS_BLK,),
    in_specs=[
        pl.BlockSpec((_NC, _ROWS_BLK, _D), lambda i: (0, i, 0)),
        pl.BlockSpec((_ROWS_BLK, _D), lambda i: (i, 0)),
        pl.BlockSpec((_D, _D), lambda i: (0, 0)),
        pl.BlockSpec((1, _D), lambda i: (0, 0)),
        pl.BlockSpec((_D, _D), lambda i: (0, 0)),
        pl.BlockSpec((1, _D), lambda i: (0, 0)),
        pl.BlockSpec((_D, _D), lambda i: (0, 0)),
        pl.BlockSpec((1, _D), lambda i: (0, 0)),
    ],
    out_specs=pl.BlockSpec((_ROWS_BLK, _D), lambda i: (i, 0)),
    out_shape=jax.ShapeDtypeStruct((_N, _D), jnp.float32),
)


def kernel(x, edge_index, W1, b1, Wh, bh, Wo, bo):
    nw = _NC * _NS
    e3 = edge_index.astype(jnp.int32).reshape(2, _E // _CHUNK, _CHUNK)
    ntail = _E - (nw - 1) * _EPT
    npad = _EPT - ntail
    pad_src = jnp.arange(npad, dtype=jnp.int32) % _N
    pad_dst = _N + (jnp.arange(npad, dtype=jnp.int32) % (_NPAD - _N))
    tail = edge_index[:, _E - ntail:].astype(jnp.int32)
    tsrc = jnp.concatenate([tail[0], pad_src]).reshape(_CPT, _CHUNK)
    tdst = jnp.concatenate([tail[1], pad_dst]).reshape(_CPT, _CHUNK)
    zeros2 = jnp.zeros((_CHUNK, _D), jnp.float32)
    zeros1 = jnp.zeros((1024,), jnp.float32)
    ones = jnp.ones((_CHUNK,), jnp.float32)

    agg2, dega, degb = _sc_agg(x, e3, tsrc, tdst, zeros2, zeros1, ones)
    dd = jnp.broadcast_to((dega + degb)[:_N, None], (_N, _D))
    y = _tc_mlp(agg2, dd,
                W1, b1.reshape(1, _D), Wh, bh.reshape(1, _D),
                Wo, bo.reshape(1, _D))
    return y
